# Initial kernel scaffold; baseline (speedup 1.0000x reference)
import jax
import jax.numpy as jnp
from jax.experimental import pallas as pl


def kernel(z, cb0, cb1, cb2):
    raise NotImplementedError("write your pallas kernel here")


# speedup vs baseline: 1.4000x; 1.4000x over previous
import functools

import jax
import jax.numpy as jnp
from jax import lax
from jax.experimental import pallas as pl
from jax.experimental.pallas import tpu as pltpu
from jax.experimental.pallas import tpu_sc as plsc

EMB = 8192
CH = 32
NTOK = 8192
TT = 128
NT = NTOK // TT
CHUNK = 128
def _I0():
    return jnp.int32(0)


_MM = (((1,), (0,)), ((), ()))


def _residual_chain(z, qs):
    r = z
    zq = None
    for q in qs:
        rq = r + (q - r)
        zq = rq if zq is None else zq + rq
        r = z - zq
    return r, zq


def _argmin_body(z_ref, *refs):
    *q_refs, cbt_ref, idx_ref = refs
    z = z_ref[...]
    r, _ = _residual_chain(z, [q[...] for q in q_refs])
    cbt = cbt_ref[...]
    a = jnp.sum(r * r, axis=1, keepdims=True)
    b = jnp.sum(cbt * cbt, axis=0, keepdims=True)
    dot = lax.dot_general(r, cbt, _MM, preferred_element_type=jnp.float32)
    dist = (a + b) - 2.0 * dot
    m = jnp.min(dist, axis=1, keepdims=True)
    ids = lax.broadcasted_iota(jnp.int32, dist.shape, 1)
    arg = jnp.min(jnp.where(dist == m, ids, jnp.int32(EMB)), axis=1)
    idx_ref[0, 0, :] = arg


def _argmin_layer(zf, qs, cbt):
    tok_spec = pl.BlockSpec((TT, CH), lambda i: (i, _I0()))
    in_specs = ([tok_spec] * (1 + len(qs))
                + [pl.BlockSpec((CH, EMB), lambda i: (_I0(), _I0()))])
    return pl.pallas_call(
        _argmin_body,
        grid=(NT,),
        in_specs=in_specs,
        out_specs=pl.BlockSpec((1, 1, TT), lambda i: (i, _I0(), _I0())),
        out_shape=jax.ShapeDtypeStruct((NT, 1, TT), jnp.int32),
    )(zf, *qs, cbt)


def _sc_gather(table, idx_flat):
    idx2 = idx_flat.reshape(NTOK // CHUNK, CHUNK)
    info = plsc.get_sparse_core_info()
    nw = info.num_cores * info.num_subcores
    bpw = NTOK // nw
    nch = bpw // CHUNK
    mesh = plsc.VectorSubcoreMesh(core_axis_name="c", subcore_axis_name="s")

    @functools.partial(
        pl.kernel,
        mesh=mesh,
        compiler_params=pltpu.CompilerParams(use_tc_tiling_on_sc=False),
        out_type=jax.ShapeDtypeStruct((NTOK, CH), jnp.float32),
        scratch_types=[
            pltpu.VMEM((CHUNK,), jnp.int32),
            pltpu.VMEM((bpw, CH), jnp.float32),
            pltpu.SemaphoreType.DMA,
        ],
    )
    def gath(table_hbm, idx_hbm, out_hbm, idx_v, rows_v, sem):
        wid = lax.axis_index("s") * info.num_cores + lax.axis_index("c")
        for j in range(nch):
            pltpu.sync_copy(idx_hbm.at[wid * nch + j], idx_v)
            pltpu.async_copy(table_hbm.at[idx_v],
                             rows_v.at[pl.ds(j * CHUNK, CHUNK)], sem).wait()
        pltpu.sync_copy(rows_v, out_hbm.at[pl.ds(wid * bpw, bpw)])

    return gath(table, idx2)


def _final_body(z_ref, q0_ref, q1_ref, q2_ref, i0_ref, i1_ref,
                zq_ref, lo_ref, loss_ref):
    z = z_ref[...]
    q0 = q0_ref[...]
    q1 = q1_ref[...]
    q2 = q2_ref[...]
    rq0 = z + (q0 - z)
    r2 = z - rq0
    rq1 = r2 + (q1 - r2)
    zq2 = rq0 + rq1
    r3 = z - zq2
    rq2 = r3 + (q2 - r3)
    zq3 = zq2 + rq2
    m0 = jnp.mean((q0 - z) ** 2)
    m1 = jnp.mean((q1 - r2) ** 2)
    m2 = jnp.mean((q2 - r3) ** 2)
    loss = (((m0 + m0) + (m1 + m1)) + (m2 + m2)) / 3.0
    loss = loss + jnp.mean((zq3 - z) ** 2)
    zq_ref[...] = zq3
    lo_ref[...] = i0_ref[...] + i1_ref[...] * EMB
    loss_ref[0, 0] = loss


def _finalize(zf, q0, q1, q2, i0, i1):
    tok = pl.BlockSpec((NTOK, CH), lambda i: (_I0(), _I0()))
    idxs = pl.BlockSpec((NT, 1, TT), lambda i: (_I0(), _I0(), _I0()))
    return pl.pallas_call(
        _final_body,
        grid=(1,),
        in_specs=[tok] * 4 + [idxs] * 2,
        out_specs=[
            pl.BlockSpec((NTOK, CH), lambda i: (_I0(), _I0())),
            pl.BlockSpec((NT, 1, TT), lambda i: (_I0(), _I0(), _I0())),
            pl.BlockSpec((1, 1), lambda i: (_I0(), _I0()),
                         memory_space=pltpu.SMEM),
        ],
        out_shape=[
            jax.ShapeDtypeStruct((NTOK, CH), jnp.float32),
            jax.ShapeDtypeStruct((NT, 1, TT), jnp.int32),
            jax.ShapeDtypeStruct((1, 1), jnp.float32),
        ],
    )(zf, q0, q1, q2, i0, i1)


def kernel(z, cb0, cb1, cb2):
    zf = z.reshape(NTOK, CH)
    i0 = _argmin_layer(zf, [], cb0.T)
    q0 = _sc_gather(cb0, i0.reshape(NTOK))
    i1 = _argmin_layer(zf, [q0], cb1.T)
    q1 = _sc_gather(cb1, i1.reshape(NTOK))
    i2 = _argmin_layer(zf, [q0, q1], cb2.T)
    q2 = _sc_gather(cb2, i2.reshape(NTOK))
    zq, lo, loss = _finalize(zf, q0, q1, q2, i0, i1)
    index_sum = (lo.reshape(8, 1024).astype(jnp.int64)
                 + i2.reshape(8, 1024).astype(jnp.int64) * (EMB * EMB))
    return zq.reshape(z.shape), index_sum, loss.reshape(())


# speedup vs baseline: 1.5461x; 1.1043x over previous
import functools

import jax
import jax.numpy as jnp
from jax import lax
from jax.experimental import pallas as pl
from jax.experimental.pallas import tpu as pltpu
from jax.experimental.pallas import tpu_sc as plsc

EMB = 8192
CH = 32
NTOK = 8192
TT = 128
NT = NTOK // TT
CHUNK = 128
def _I0():
    return jnp.int32(0)


_MM = (((1,), (0,)), ((), ()))


def _residual_chain(z, qs):
    r = z
    zq = None
    for q in qs:
        rq = r + (q - r)
        zq = rq if zq is None else zq + rq
        r = z - zq
    return r, zq


def _argmin_body(z_ref, *refs):
    *q_refs, cbt_ref, idx_ref, b_ref, ids_ref = refs
    z = z_ref[...]
    r, _ = _residual_chain(z, [q[...] for q in q_refs])
    cbt = cbt_ref[...]

    @pl.when(pl.program_id(0) == 0)
    def _():
        b_ref[...] = jnp.sum(cbt * cbt, axis=0, keepdims=True)
        ids_ref[...] = lax.broadcasted_iota(
            jnp.int32, (1, EMB), 1).astype(jnp.float32)

    a = jnp.sum(r * r, axis=1, keepdims=True)
    b = b_ref[...]
    dot = lax.dot_general(r, cbt, _MM, preferred_element_type=jnp.float32)
    dist = (a + b) - 2.0 * dot
    m = jnp.min(dist, axis=1, keepdims=True)
    ids = ids_ref[...]
    arg = jnp.min(jnp.where(dist == m, ids, jnp.float32(EMB)), axis=1)
    idx_ref[0, 0, :] = arg.astype(jnp.int32)


def _argmin_layer(zf, qs, cbt):
    tok_spec = pl.BlockSpec((TT, CH), lambda i: (i, _I0()))
    in_specs = ([tok_spec] * (1 + len(qs))
                + [pl.BlockSpec((CH, EMB), lambda i: (_I0(), _I0()))])
    return pl.pallas_call(
        _argmin_body,
        grid=(NT,),
        in_specs=in_specs,
        out_specs=pl.BlockSpec((1, 1, TT), lambda i: (i, _I0(), _I0())),
        out_shape=jax.ShapeDtypeStruct((NT, 1, TT), jnp.int32),
        scratch_shapes=[pltpu.VMEM((1, EMB), jnp.float32),
                        pltpu.VMEM((1, EMB), jnp.float32)],
    )(zf, *qs, cbt)


def _sc_gather(table, idx_flat):
    idx2 = idx_flat.reshape(NTOK // CHUNK, CHUNK)
    info = plsc.get_sparse_core_info()
    nw = info.num_cores * info.num_subcores
    bpw = NTOK // nw
    nch = bpw // CHUNK
    mesh = plsc.VectorSubcoreMesh(core_axis_name="c", subcore_axis_name="s")

    @functools.partial(
        pl.kernel,
        mesh=mesh,
        compiler_params=pltpu.CompilerParams(use_tc_tiling_on_sc=False),
        out_type=jax.ShapeDtypeStruct((NTOK, CH), jnp.float32),
        scratch_types=[
            pltpu.VMEM((CHUNK,), jnp.int32),
            pltpu.VMEM((bpw, CH), jnp.float32),
            pltpu.SemaphoreType.DMA,
        ],
    )
    def gath(table_hbm, idx_hbm, out_hbm, idx_v, rows_v, sem):
        wid = lax.axis_index("s") * info.num_cores + lax.axis_index("c")
        for j in range(nch):
            pltpu.sync_copy(idx_hbm.at[wid * nch + j], idx_v)
            pltpu.async_copy(table_hbm.at[idx_v],
                             rows_v.at[pl.ds(j * CHUNK, CHUNK)], sem).wait()
        pltpu.sync_copy(rows_v, out_hbm.at[pl.ds(wid * bpw, bpw)])

    return gath(table, idx2)


def _final_body(z_ref, q0_ref, q1_ref, q2_ref, i0_ref, i1_ref,
                zq_ref, lo_ref, loss_ref):
    z = z_ref[...]
    q0 = q0_ref[...]
    q1 = q1_ref[...]
    q2 = q2_ref[...]
    rq0 = z + (q0 - z)
    r2 = z - rq0
    rq1 = r2 + (q1 - r2)
    zq2 = rq0 + rq1
    r3 = z - zq2
    rq2 = r3 + (q2 - r3)
    zq3 = zq2 + rq2
    m0 = jnp.mean((q0 - z) ** 2)
    m1 = jnp.mean((q1 - r2) ** 2)
    m2 = jnp.mean((q2 - r3) ** 2)
    loss = (((m0 + m0) + (m1 + m1)) + (m2 + m2)) / 3.0
    loss = loss + jnp.mean((zq3 - z) ** 2)
    zq_ref[...] = zq3
    lo_ref[...] = i0_ref[...] + i1_ref[...] * EMB
    loss_ref[0, 0] = loss


def _finalize(zf, q0, q1, q2, i0, i1):
    tok = pl.BlockSpec((NTOK, CH), lambda i: (_I0(), _I0()))
    idxs = pl.BlockSpec((NT, 1, TT), lambda i: (_I0(), _I0(), _I0()))
    return pl.pallas_call(
        _final_body,
        grid=(1,),
        in_specs=[tok] * 4 + [idxs] * 2,
        out_specs=[
            pl.BlockSpec((NTOK, CH), lambda i: (_I0(), _I0())),
            pl.BlockSpec((NT, 1, TT), lambda i: (_I0(), _I0(), _I0())),
            pl.BlockSpec((1, 1), lambda i: (_I0(), _I0()),
                         memory_space=pltpu.SMEM),
        ],
        out_shape=[
            jax.ShapeDtypeStruct((NTOK, CH), jnp.float32),
            jax.ShapeDtypeStruct((NT, 1, TT), jnp.int32),
            jax.ShapeDtypeStruct((1, 1), jnp.float32),
        ],
    )(zf, q0, q1, q2, i0, i1)


def kernel(z, cb0, cb1, cb2):
    zf = z.reshape(NTOK, CH)
    i0 = _argmin_layer(zf, [], cb0.T)
    q0 = _sc_gather(cb0, i0.reshape(NTOK))
    i1 = _argmin_layer(zf, [q0], cb1.T)
    q1 = _sc_gather(cb1, i1.reshape(NTOK))
    i2 = _argmin_layer(zf, [q0, q1], cb2.T)
    q2 = _sc_gather(cb2, i2.reshape(NTOK))
    zq, lo, loss = _finalize(zf, q0, q1, q2, i0, i1)
    index_sum = (lo.reshape(8, 1024).astype(jnp.int64)
                 + i2.reshape(8, 1024).astype(jnp.int64) * (EMB * EMB))
    return zq.reshape(z.shape), index_sum, loss.reshape(())


# speedup vs baseline: 1.6896x; 1.0928x over previous
import functools

import jax
import jax.numpy as jnp
from jax import lax
from jax.experimental import pallas as pl
from jax.experimental.pallas import tpu as pltpu
from jax.experimental.pallas import tpu_sc as plsc

EMB = 8192
CH = 32
NTOK = 8192
TT = 256
NT = NTOK // TT
CHUNK = 128
def _I0():
    return jnp.int32(0)


_MM = (((1,), (0,)), ((), ()))


def _residual_chain(z, qs):
    r = z
    zq = None
    for q in qs:
        rq = r + (q - r)
        zq = rq if zq is None else zq + rq
        r = z - zq
    return r, zq


def _argmin_body(z_ref, *refs):
    *q_refs, cbt_ref, idx_ref, b_ref, ids_ref = refs
    z = z_ref[...]
    r, _ = _residual_chain(z, [q[...] for q in q_refs])
    cbt = cbt_ref[...]

    @pl.when(pl.program_id(0) == 0)
    def _():
        b_ref[...] = jnp.sum(cbt * cbt, axis=0, keepdims=True)
        ids_ref[...] = lax.broadcasted_iota(
            jnp.int32, (1, EMB), 1).astype(jnp.float32)

    a = jnp.sum(r * r, axis=1, keepdims=True)
    b = b_ref[...]
    dot = lax.dot_general(r, cbt, _MM, preferred_element_type=jnp.float32)
    dist = (a + b) - 2.0 * dot
    m = jnp.min(dist, axis=1, keepdims=True)
    ids = ids_ref[...]
    arg = jnp.min(jnp.where(dist == m, ids, jnp.float32(EMB)), axis=1)
    idx_ref[0, 0, :] = arg.astype(jnp.int32)


def _argmin_layer(zf, qs, cbt):
    tok_spec = pl.BlockSpec((TT, CH), lambda i: (i, _I0()))
    in_specs = ([tok_spec] * (1 + len(qs))
                + [pl.BlockSpec((CH, EMB), lambda i: (_I0(), _I0()))])
    return pl.pallas_call(
        _argmin_body,
        grid=(NT,),
        in_specs=in_specs,
        out_specs=pl.BlockSpec((1, 1, TT), lambda i: (i, _I0(), _I0())),
        out_shape=jax.ShapeDtypeStruct((NT, 1, TT), jnp.int32),
        scratch_shapes=[pltpu.VMEM((1, EMB), jnp.float32),
                        pltpu.VMEM((1, EMB), jnp.float32)],
    )(zf, *qs, cbt)


def _sc_gather(table, idx_flat):
    idx2 = idx_flat.reshape(NTOK // CHUNK, CHUNK)
    info = plsc.get_sparse_core_info()
    nw = info.num_cores * info.num_subcores
    bpw = NTOK // nw
    nch = bpw // CHUNK
    mesh = plsc.VectorSubcoreMesh(core_axis_name="c", subcore_axis_name="s")

    @functools.partial(
        pl.kernel,
        mesh=mesh,
        compiler_params=pltpu.CompilerParams(use_tc_tiling_on_sc=False),
        out_type=jax.ShapeDtypeStruct((NTOK, CH), jnp.float32),
        scratch_types=[
            pltpu.VMEM((CHUNK,), jnp.int32),
            pltpu.VMEM((bpw, CH), jnp.float32),
            pltpu.SemaphoreType.DMA,
        ],
    )
    def gath(table_hbm, idx_hbm, out_hbm, idx_v, rows_v, sem):
        wid = lax.axis_index("s") * info.num_cores + lax.axis_index("c")
        for j in range(nch):
            pltpu.sync_copy(idx_hbm.at[wid * nch + j], idx_v)
            pltpu.async_copy(table_hbm.at[idx_v],
                             rows_v.at[pl.ds(j * CHUNK, CHUNK)], sem).wait()
        pltpu.sync_copy(rows_v, out_hbm.at[pl.ds(wid * bpw, bpw)])

    return gath(table, idx2)


def _final_body(z_ref, q0_ref, q1_ref, q2_ref, i0_ref, i1_ref,
                zq_ref, lo_ref, loss_ref):
    z = z_ref[...]
    q0 = q0_ref[...]
    q1 = q1_ref[...]
    q2 = q2_ref[...]
    rq0 = z + (q0 - z)
    r2 = z - rq0
    rq1 = r2 + (q1 - r2)
    zq2 = rq0 + rq1
    r3 = z - zq2
    rq2 = r3 + (q2 - r3)
    zq3 = zq2 + rq2
    m0 = jnp.mean((q0 - z) ** 2)
    m1 = jnp.mean((q1 - r2) ** 2)
    m2 = jnp.mean((q2 - r3) ** 2)
    loss = (((m0 + m0) + (m1 + m1)) + (m2 + m2)) / 3.0
    loss = loss + jnp.mean((zq3 - z) ** 2)
    zq_ref[...] = zq3
    lo_ref[...] = i0_ref[...] + i1_ref[...] * EMB
    loss_ref[0, 0] = loss


def _finalize(zf, q0, q1, q2, i0, i1):
    tok = pl.BlockSpec((NTOK, CH), lambda i: (_I0(), _I0()))
    idxs = pl.BlockSpec((NT, 1, TT), lambda i: (_I0(), _I0(), _I0()))
    return pl.pallas_call(
        _final_body,
        grid=(1,),
        in_specs=[tok] * 4 + [idxs] * 2,
        out_specs=[
            pl.BlockSpec((NTOK, CH), lambda i: (_I0(), _I0())),
            pl.BlockSpec((NT, 1, TT), lambda i: (_I0(), _I0(), _I0())),
            pl.BlockSpec((1, 1), lambda i: (_I0(), _I0()),
                         memory_space=pltpu.SMEM),
        ],
        out_shape=[
            jax.ShapeDtypeStruct((NTOK, CH), jnp.float32),
            jax.ShapeDtypeStruct((NT, 1, TT), jnp.int32),
            jax.ShapeDtypeStruct((1, 1), jnp.float32),
        ],
    )(zf, q0, q1, q2, i0, i1)


def kernel(z, cb0, cb1, cb2):
    zf = z.reshape(NTOK, CH)
    i0 = _argmin_layer(zf, [], cb0.T)
    q0 = _sc_gather(cb0, i0.reshape(NTOK))
    i1 = _argmin_layer(zf, [q0], cb1.T)
    q1 = _sc_gather(cb1, i1.reshape(NTOK))
    i2 = _argmin_layer(zf, [q0, q1], cb2.T)
    q2 = _sc_gather(cb2, i2.reshape(NTOK))
    zq, lo, loss = _finalize(zf, q0, q1, q2, i0, i1)
    index_sum = (lo.reshape(8, 1024).astype(jnp.int64)
                 + i2.reshape(8, 1024).astype(jnp.int64) * (EMB * EMB))
    return zq.reshape(z.shape), index_sum, loss.reshape(())


# speedup vs baseline: 1.7947x; 1.0622x over previous
import functools

import jax
import jax.numpy as jnp
from jax import lax
from jax.experimental import pallas as pl
from jax.experimental.pallas import tpu as pltpu
from jax.experimental.pallas import tpu_sc as plsc

EMB = 8192
CH = 32
NTOK = 8192
TT = 512
NT = NTOK // TT
CHUNK = 128
def _I0():
    return jnp.int32(0)


_MM = (((1,), (0,)), ((), ()))


def _residual_chain(z, qs):
    r = z
    zq = None
    for q in qs:
        rq = r + (q - r)
        zq = rq if zq is None else zq + rq
        r = z - zq
    return r, zq


def _argmin_body(z_ref, *refs):
    *q_refs, cbt_ref, idx_ref, b_ref, ids_ref = refs
    z = z_ref[...]
    r, _ = _residual_chain(z, [q[...] for q in q_refs])
    cbt = cbt_ref[...]

    @pl.when(pl.program_id(0) == 0)
    def _():
        b_ref[...] = jnp.sum(cbt * cbt, axis=0, keepdims=True)
        ids_ref[...] = lax.broadcasted_iota(
            jnp.int32, (1, EMB), 1).astype(jnp.float32)

    a = jnp.sum(r * r, axis=1, keepdims=True)
    b = b_ref[...]
    dot = lax.dot_general(r, cbt, _MM, preferred_element_type=jnp.float32)
    dist = (a + b) - 2.0 * dot
    m = jnp.min(dist, axis=1, keepdims=True)
    ids = ids_ref[...]
    arg = jnp.min(jnp.where(dist == m, ids, jnp.float32(EMB)), axis=1)
    idx_ref[0, 0, :] = arg.astype(jnp.int32)


def _argmin_layer(zf, qs, cbt):
    tok_spec = pl.BlockSpec((TT, CH), lambda i: (i, _I0()))
    in_specs = ([tok_spec] * (1 + len(qs))
                + [pl.BlockSpec((CH, EMB), lambda i: (_I0(), _I0()))])
    return pl.pallas_call(
        _argmin_body,
        grid=(NT,),
        in_specs=in_specs,
        out_specs=pl.BlockSpec((1, 1, TT), lambda i: (i, _I0(), _I0())),
        out_shape=jax.ShapeDtypeStruct((NT, 1, TT), jnp.int32),
        scratch_shapes=[pltpu.VMEM((1, EMB), jnp.float32),
                        pltpu.VMEM((1, EMB), jnp.float32)],
    )(zf, *qs, cbt)


def _sc_gather(table, idx_flat):
    idx2 = idx_flat.reshape(NTOK // CHUNK, CHUNK)
    info = plsc.get_sparse_core_info()
    nw = info.num_cores * info.num_subcores
    bpw = NTOK // nw
    nch = bpw // CHUNK
    mesh = plsc.VectorSubcoreMesh(core_axis_name="c", subcore_axis_name="s")

    @functools.partial(
        pl.kernel,
        mesh=mesh,
        compiler_params=pltpu.CompilerParams(use_tc_tiling_on_sc=False),
        out_type=jax.ShapeDtypeStruct((NTOK, CH), jnp.float32),
        scratch_types=[
            pltpu.VMEM((CHUNK,), jnp.int32),
            pltpu.VMEM((bpw, CH), jnp.float32),
            pltpu.SemaphoreType.DMA,
        ],
    )
    def gath(table_hbm, idx_hbm, out_hbm, idx_v, rows_v, sem):
        wid = lax.axis_index("s") * info.num_cores + lax.axis_index("c")
        for j in range(nch):
            pltpu.sync_copy(idx_hbm.at[wid * nch + j], idx_v)
            pltpu.async_copy(table_hbm.at[idx_v],
                             rows_v.at[pl.ds(j * CHUNK, CHUNK)], sem).wait()
        pltpu.sync_copy(rows_v, out_hbm.at[pl.ds(wid * bpw, bpw)])

    return gath(table, idx2)


def _final_body(z_ref, q0_ref, q1_ref, q2_ref, i0_ref, i1_ref,
                zq_ref, lo_ref, loss_ref):
    z = z_ref[...]
    q0 = q0_ref[...]
    q1 = q1_ref[...]
    q2 = q2_ref[...]
    rq0 = z + (q0 - z)
    r2 = z - rq0
    rq1 = r2 + (q1 - r2)
    zq2 = rq0 + rq1
    r3 = z - zq2
    rq2 = r3 + (q2 - r3)
    zq3 = zq2 + rq2
    m0 = jnp.mean((q0 - z) ** 2)
    m1 = jnp.mean((q1 - r2) ** 2)
    m2 = jnp.mean((q2 - r3) ** 2)
    loss = (((m0 + m0) + (m1 + m1)) + (m2 + m2)) / 3.0
    loss = loss + jnp.mean((zq3 - z) ** 2)
    zq_ref[...] = zq3
    lo_ref[...] = i0_ref[...] + i1_ref[...] * EMB
    loss_ref[0, 0] = loss


def _finalize(zf, q0, q1, q2, i0, i1):
    tok = pl.BlockSpec((NTOK, CH), lambda i: (_I0(), _I0()))
    idxs = pl.BlockSpec((NT, 1, TT), lambda i: (_I0(), _I0(), _I0()))
    return pl.pallas_call(
        _final_body,
        grid=(1,),
        in_specs=[tok] * 4 + [idxs] * 2,
        out_specs=[
            pl.BlockSpec((NTOK, CH), lambda i: (_I0(), _I0())),
            pl.BlockSpec((NT, 1, TT), lambda i: (_I0(), _I0(), _I0())),
            pl.BlockSpec((1, 1), lambda i: (_I0(), _I0()),
                         memory_space=pltpu.SMEM),
        ],
        out_shape=[
            jax.ShapeDtypeStruct((NTOK, CH), jnp.float32),
            jax.ShapeDtypeStruct((NT, 1, TT), jnp.int32),
            jax.ShapeDtypeStruct((1, 1), jnp.float32),
        ],
    )(zf, q0, q1, q2, i0, i1)


def kernel(z, cb0, cb1, cb2):
    zf = z.reshape(NTOK, CH)
    i0 = _argmin_layer(zf, [], cb0.T)
    q0 = _sc_gather(cb0, i0.reshape(NTOK))
    i1 = _argmin_layer(zf, [q0], cb1.T)
    q1 = _sc_gather(cb1, i1.reshape(NTOK))
    i2 = _argmin_layer(zf, [q0, q1], cb2.T)
    q2 = _sc_gather(cb2, i2.reshape(NTOK))
    zq, lo, loss = _finalize(zf, q0, q1, q2, i0, i1)
    index_sum = (lo.reshape(8, 1024).astype(jnp.int64)
                 + i2.reshape(8, 1024).astype(jnp.int64) * (EMB * EMB))
    return zq.reshape(z.shape), index_sum, loss.reshape(())


# speedup vs baseline: 1.7999x; 1.0029x over previous
import functools

import jax
import jax.numpy as jnp
from jax import lax
from jax.experimental import pallas as pl
from jax.experimental.pallas import tpu as pltpu
from jax.experimental.pallas import tpu_sc as plsc

EMB = 8192
CH = 32
B0 = 8
B1 = 1024
NTOK = B0 * B1
TT = 512
NT = NTOK // TT
TPB = B1 // TT
CHUNK = 128
FT = 1024
NF = NTOK // FT
_MM = (((1,), (0,)), ((), ()))


def _I0():
    return jnp.int32(0)


def _residual_chain(z, qs):
    r = z
    zq = None
    for q in qs:
        rq = r + (q - r)
        zq = rq if zq is None else zq + rq
        r = z - zq
    return r, zq


def _argmin_body(z_ref, *refs):
    *q_refs, cbt_ref, idx_ref, b_ref, ids_ref = refs
    z = z_ref[0]
    r, _ = _residual_chain(z, [q[...] for q in q_refs])
    cbt = cbt_ref[...]

    @pl.when(pl.program_id(0) == 0)
    def _():
        b_ref[...] = jnp.sum(cbt * cbt, axis=0, keepdims=True)
        ids_ref[...] = lax.broadcasted_iota(
            jnp.int32, (1, EMB), 1).astype(jnp.float32)

    a = jnp.sum(r * r, axis=1, keepdims=True)
    b = b_ref[...]
    dot = lax.dot_general(r, cbt, _MM, preferred_element_type=jnp.float32)
    dist = (a + b) - 2.0 * dot
    m = jnp.min(dist, axis=1, keepdims=True)
    ids = ids_ref[...]
    arg = jnp.min(jnp.where(dist == m, ids, jnp.float32(EMB)), axis=1)
    idx_ref[0, 0, :] = arg.astype(jnp.int32)


def _argmin_layer(z, qs, cbt):
    z_spec = pl.BlockSpec((1, TT, CH),
                          lambda i: (i // TPB, i % TPB, _I0()))
    tok_spec = pl.BlockSpec((TT, CH), lambda i: (i, _I0()))
    in_specs = ([z_spec] + [tok_spec] * len(qs)
                + [pl.BlockSpec((CH, EMB), lambda i: (_I0(), _I0()))])
    return pl.pallas_call(
        _argmin_body,
        grid=(NT,),
        in_specs=in_specs,
        out_specs=pl.BlockSpec((1, 1, TT), lambda i: (i, _I0(), _I0())),
        out_shape=jax.ShapeDtypeStruct((NT, 1, TT), jnp.int32),
        scratch_shapes=[pltpu.VMEM((1, EMB), jnp.float32),
                        pltpu.VMEM((1, EMB), jnp.float32)],
    )(z, *qs, cbt)


def _sc_gather(table, idx):
    info = plsc.get_sparse_core_info()
    nw = info.num_cores * info.num_subcores
    bpw = NTOK // nw
    nch = bpw // CHUNK
    cpt = TT // CHUNK
    mesh = plsc.VectorSubcoreMesh(core_axis_name="c", subcore_axis_name="s")

    @functools.partial(
        pl.kernel,
        mesh=mesh,
        compiler_params=pltpu.CompilerParams(use_tc_tiling_on_sc=False),
        out_type=jax.ShapeDtypeStruct((NTOK, CH), jnp.float32),
        scratch_types=[
            pltpu.VMEM((CHUNK,), jnp.int32),
            pltpu.VMEM((bpw, CH), jnp.float32),
            pltpu.SemaphoreType.DMA,
        ],
    )
    def gath(table_hbm, idx_hbm, out_hbm, idx_v, rows_v, sem):
        wid = lax.axis_index("s") * jnp.int32(info.num_cores) + lax.axis_index("c")
        c0 = wid * jnp.int32(nch)
        for j in range(nch):
            c = c0 + jnp.int32(j)
            pltpu.sync_copy(
                idx_hbm.at[c // jnp.int32(cpt), _I0(),
                           pl.ds((c % jnp.int32(cpt)) * jnp.int32(CHUNK), CHUNK)],
                idx_v)
            pltpu.async_copy(table_hbm.at[idx_v],
                             rows_v.at[pl.ds(j * CHUNK, CHUNK)], sem).wait()
        pltpu.sync_copy(rows_v, out_hbm.at[pl.ds(wid * jnp.int32(bpw), bpw)])

    return gath(table, idx)


def _final_body(z_ref, q0_ref, q1_ref, q2_ref, i0_ref, i1_ref,
                zq_ref, lo_ref, loss_ref, acc_ref):
    z = z_ref[0]
    q0 = q0_ref[...]
    q1 = q1_ref[...]
    q2 = q2_ref[...]
    rq0 = z + (q0 - z)
    r2 = z - rq0
    rq1 = r2 + (q1 - r2)
    zq2 = rq0 + rq1
    r3 = z - zq2
    rq2 = r3 + (q2 - r3)
    zq3 = zq2 + rq2

    @pl.when(pl.program_id(0) == 0)
    def _():
        for k in range(4):
            acc_ref[k] = jnp.float32(0.0)

    acc_ref[0] += jnp.sum((q0 - z) ** 2)
    acc_ref[1] += jnp.sum((q1 - r2) ** 2)
    acc_ref[2] += jnp.sum((q2 - r3) ** 2)
    acc_ref[3] += jnp.sum((zq3 - z) ** 2)
    zq_ref[0] = zq3
    lo_ref[...] = i0_ref[...] + i1_ref[...] * EMB

    @pl.when(pl.program_id(0) == NF - 1)
    def _():
        inv = jnp.float32(1.0 / (NTOK * CH))
        m0 = acc_ref[0] * inv
        m1 = acc_ref[1] * inv
        m2 = acc_ref[2] * inv
        loss = (((m0 + m0) + (m1 + m1)) + (m2 + m2)) / 3.0
        loss_ref[0, 0] = loss + acc_ref[3] * inv


def _finalize(z, q0, q1, q2, i0, i1):
    z_spec = pl.BlockSpec((1, FT, CH), lambda i: (i, _I0(), _I0()))
    tok = pl.BlockSpec((FT, CH), lambda i: (i, _I0()))
    rpf = FT // TT
    idxs = pl.BlockSpec((rpf, 1, TT), lambda i: (i, _I0(), _I0()))
    return pl.pallas_call(
        _final_body,
        grid=(NF,),
        in_specs=[z_spec] + [tok] * 3 + [idxs] * 2,
        out_specs=[
            z_spec,
            idxs,
            pl.BlockSpec((1, 1), lambda i: (_I0(), _I0()),
                         memory_space=pltpu.SMEM),
        ],
        out_shape=[
            jax.ShapeDtypeStruct((B0, B1, CH), jnp.float32),
            jax.ShapeDtypeStruct((NT, 1, TT), jnp.int32),
            jax.ShapeDtypeStruct((1, 1), jnp.float32),
        ],
        scratch_shapes=[pltpu.SMEM((4,), jnp.float32)],
    )(z, q0, q1, q2, i0, i1)


def kernel(z, cb0, cb1, cb2):
    i0 = _argmin_layer(z, [], cb0.T)
    q0 = _sc_gather(cb0, i0)
    i1 = _argmin_layer(z, [q0], cb1.T)
    q1 = _sc_gather(cb1, i1)
    i2 = _argmin_layer(z, [q0, q1], cb2.T)
    q2 = _sc_gather(cb2, i2)
    zq, lo, loss = _finalize(z, q0, q1, q2, i0, i1)
    index_sum = (lo.reshape(B0, B1).astype(jnp.int64)
                 + i2.reshape(B0, B1).astype(jnp.int64) * (EMB * EMB))
    return zq, index_sum, loss.reshape(())
